# baseline (speedup 1.0000x reference)
import dataclasses
import functools
import math

import jax
import jax.numpy as jnp
from jax import lax
from jax.experimental import pallas as pl
from jax.experimental.pallas import tpu as pltpu
from jax.experimental.pallas import tpu_sc as plsc

_INV_TEMP = 1.0 / math.e
_ACTIVE_RATIO = 0.7


def _gate_block(x_ref, w_ref, o_ref):
    o_ref[...] = jax.lax.dot_general(
        x_ref[...], w_ref[...],
        (((1,), (1,)), ((), ())),
        preferred_element_type=jnp.float32,
    ) * _INV_TEMP


def _merge_lo(a, b):
    return jnp.sort(jnp.minimum(a, lax.rev(b, (0,))))


def _merge_hi(a, b):
    return jnp.sort(jnp.maximum(a, lax.rev(b, (0,))))


def _sc_route(N, E, C, n_drop):
    NW = 32
    per_w = N // NW
    n_chunks = per_w // C
    mesh = plsc.VectorSubcoreMesh(
        core_axis_name="c", subcore_axis_name="s",
        num_cores=2, num_subcores=16,
    )

    cp = pltpu.CompilerParams()
    if "needs_layout_passes" in pltpu.CompilerParams.__dataclass_fields__:
        cp = dataclasses.replace(cp, needs_layout_passes=False)

    @functools.partial(
        pl.kernel,
        out_type=jax.ShapeDtypeStruct((N, E), jnp.float32),
        mesh=mesh,
        compiler_params=cp,
        scratch_types=[
            pltpu.VMEM((C, E), jnp.float32),
            pltpu.VMEM((C, E), jnp.float32),
            pltpu.SemaphoreType.DMA,
        ],
    )
    def route(s_hbm, o_hbm, s_v, o_v, sem):
        wid = lax.axis_index("s") * 2 + lax.axis_index("c")
        base = wid * per_w

        @pl.loop(0, n_chunks)
        def _(ci):
            off = base + ci * C
            pltpu.sync_copy(s_hbm.at[pl.ds(off, C)], s_v)

            @pl.loop(0, C, unroll=2)
            def _(t):
                s0 = s_v[t, 0:16]
                s1 = s_v[t, 16:32]
                s2 = s_v[t, 32:48]
                s3 = s_v[t, 48:64]
                v0 = jnp.sort(s0)
                v1 = jnp.sort(s1)
                v2 = jnp.sort(s2)
                v3 = jnp.sort(s3)
                lo01 = _merge_lo(v0, v1)
                hi01 = _merge_hi(v0, v1)
                lo23 = _merge_lo(v2, v3)
                hi23 = _merge_hi(v2, v3)
                h1 = _merge_hi(lo01, lo23)
                l2 = _merge_lo(h1, hi01)
                l3 = _merge_lo(l2, hi23)
                thr = l3[n_drop - 16]
                e0 = jnp.where(s0 >= thr, jnp.exp(s0), 0.0)
                e1 = jnp.where(s1 >= thr, jnp.exp(s1), 0.0)
                e2 = jnp.where(s2 >= thr, jnp.exp(s2), 0.0)
                e3 = jnp.where(s3 >= thr, jnp.exp(s3), 0.0)
                tot = jnp.sum(e0 + e1 + e2 + e3, axis=0)
                o_v[t, 0:16] = e0 / tot
                o_v[t, 16:32] = e1 / tot
                o_v[t, 32:48] = e2 / tot
                o_v[t, 48:64] = e3 / tot

            pltpu.sync_copy(o_v, o_hbm.at[pl.ds(off, C)])

    return route


def kernel(x, W):
    B, S, H = x.shape
    E = W.shape[0]
    N = B * S
    n_drop = E - max(1, int(E * _ACTIVE_RATIO))
    T = 512
    xf = x.reshape(N, H)

    scores = pl.pallas_call(
        _gate_block,
        grid=(N // T,),
        in_specs=[
            pl.BlockSpec((T, H), lambda i: (i, 0)),
            pl.BlockSpec((E, H), lambda i: (0, 0)),
        ],
        out_specs=pl.BlockSpec((T, E), lambda i: (i, 0)),
        out_shape=jax.ShapeDtypeStruct((N, E), jnp.float32),
        compiler_params=pltpu.CompilerParams(
            dimension_semantics=("arbitrary",),
        ),
    )(xf, W)

    return _sc_route(N, E, 256, n_drop)(scores).reshape(B, S, E)


# speedup vs baseline: 1.5096x; 1.5096x over previous
import dataclasses
import functools
import math

import jax
import jax.numpy as jnp
from jax import lax
from jax.experimental import pallas as pl
from jax.experimental.pallas import tpu as pltpu
from jax.experimental.pallas import tpu_sc as plsc

_INV_TEMP = 1.0 / math.e
_ACTIVE_RATIO = 0.7


def _gate_block(x_ref, w_ref, o_ref):
    o_ref[...] = jax.lax.dot_general(
        x_ref[...], w_ref[...],
        (((1,), (1,)), ((), ())),
        preferred_element_type=jnp.float32,
    ) * _INV_TEMP


def _merge_lo(a, b):
    return jnp.sort(jnp.minimum(a, lax.rev(b, (0,))))


def _merge_hi(a, b):
    return jnp.sort(jnp.maximum(a, lax.rev(b, (0,))))


def _sc_route(N, E, C, n_drop):
    NW = 32
    per_w = N // NW
    n_chunks = per_w // C
    mesh = plsc.VectorSubcoreMesh(
        core_axis_name="c", subcore_axis_name="s",
        num_cores=2, num_subcores=16,
    )

    cp = pltpu.CompilerParams()
    if "needs_layout_passes" in pltpu.CompilerParams.__dataclass_fields__:
        cp = dataclasses.replace(cp, needs_layout_passes=False)

    @functools.partial(
        pl.kernel,
        out_type=jax.ShapeDtypeStruct((N, E), jnp.float32),
        mesh=mesh,
        compiler_params=cp,
        scratch_types=[
            pltpu.VMEM((C, E), jnp.float32),
            pltpu.VMEM((C, E), jnp.float32),
            pltpu.SemaphoreType.DMA,
        ],
    )
    def route(s_hbm, o_hbm, s_v, o_v, sem):
        wid = lax.axis_index("s") * 2 + lax.axis_index("c")
        base = wid * per_w

        @pl.loop(0, n_chunks)
        def _(ci):
            off = base + ci * C
            pltpu.sync_copy(s_hbm.at[pl.ds(off, C)], s_v)

            @pl.loop(0, C)
            def _(t):
                s0 = s_v[t, 0:16]
                s1 = s_v[t, 16:32]
                s2 = s_v[t, 32:48]
                s3 = s_v[t, 48:64]
                v0 = jnp.sort(s0)
                v1 = jnp.sort(s1)
                v2 = jnp.sort(s2)
                v3 = jnp.sort(s3)
                lo01 = _merge_lo(v0, v1)
                hi01 = _merge_hi(v0, v1)
                lo23 = _merge_lo(v2, v3)
                hi23 = _merge_hi(v2, v3)
                h1 = _merge_hi(lo01, lo23)
                l2 = _merge_lo(h1, hi01)
                l3 = _merge_lo(l2, hi23)
                thr = l3[n_drop - 16]
                e0 = jnp.where(s0 >= thr, jnp.exp(s0), 0.0)
                e1 = jnp.where(s1 >= thr, jnp.exp(s1), 0.0)
                e2 = jnp.where(s2 >= thr, jnp.exp(s2), 0.0)
                e3 = jnp.where(s3 >= thr, jnp.exp(s3), 0.0)
                tot = jnp.sum(e0 + e1 + e2 + e3, axis=0)
                o_v[t, 0:16] = e0 / tot
                o_v[t, 16:32] = e1 / tot
                o_v[t, 32:48] = e2 / tot
                o_v[t, 48:64] = e3 / tot

            pltpu.sync_copy(o_v, o_hbm.at[pl.ds(off, C)])

    return route


def kernel(x, W):
    B, S, H = x.shape
    E = W.shape[0]
    N = B * S
    n_drop = E - max(1, int(E * _ACTIVE_RATIO))
    T = 512
    xf = x.reshape(N, H)

    scores = pl.pallas_call(
        _gate_block,
        grid=(N // T,),
        in_specs=[
            pl.BlockSpec((T, H), lambda i: (i, 0)),
            pl.BlockSpec((E, H), lambda i: (0, 0)),
        ],
        out_specs=pl.BlockSpec((T, E), lambda i: (i, 0)),
        out_shape=jax.ShapeDtypeStruct((N, E), jnp.float32),
        compiler_params=pltpu.CompilerParams(
            dimension_semantics=("arbitrary",),
        ),
    )(xf, W)

    return _sc_route(N, E, 256, n_drop)(scores).reshape(B, S, E)


# speedup vs baseline: 1.6185x; 1.0721x over previous
import dataclasses
import functools
import math

import jax
import jax.numpy as jnp
from jax import lax
from jax.experimental import pallas as pl
from jax.experimental.pallas import tpu as pltpu
from jax.experimental.pallas import tpu_sc as plsc

_INV_TEMP = 1.0 / math.e
_ACTIVE_RATIO = 0.7


def _gate_block(x_ref, w_ref, o_ref):
    o_ref[...] = jax.lax.dot_general(
        x_ref[...], w_ref[...],
        (((1,), (1,)), ((), ())),
        preferred_element_type=jnp.float32,
    ) * _INV_TEMP


def _merge_lo(a, b):
    return jnp.sort(jnp.minimum(a, lax.rev(b, (0,))))


def _merge_hi(a, b):
    return jnp.sort(jnp.maximum(a, lax.rev(b, (0,))))


def _sc_route(N, E, C, n_drop):
    NW = 32
    per_w = N // NW
    n_chunks = per_w // C
    mesh = plsc.VectorSubcoreMesh(
        core_axis_name="c", subcore_axis_name="s",
        num_cores=2, num_subcores=16,
    )

    cp = pltpu.CompilerParams()
    if "needs_layout_passes" in pltpu.CompilerParams.__dataclass_fields__:
        cp = dataclasses.replace(cp, needs_layout_passes=False)

    @functools.partial(
        pl.kernel,
        out_type=jax.ShapeDtypeStruct((N, E), jnp.float32),
        mesh=mesh,
        compiler_params=cp,
        scratch_types=[
            pltpu.VMEM((2, C, E), jnp.float32),
            pltpu.VMEM((2, C, E), jnp.float32),
            pltpu.SemaphoreType.DMA((2,)),
            pltpu.SemaphoreType.DMA((2,)),
        ],
    )
    def route(s_hbm, o_hbm, s_v, o_v, in_sem, out_sem):
        wid = lax.axis_index("s") * 2 + lax.axis_index("c")
        base = wid * per_w

        def start_in(ci, slot):
            return pltpu.async_copy(
                s_hbm.at[pl.ds(base + ci * C, C)], s_v.at[slot], in_sem.at[slot])

        def start_out(ci, slot):
            return pltpu.async_copy(
                o_v.at[slot], o_hbm.at[pl.ds(base + ci * C, C)], out_sem.at[slot])

        copies_in = [start_in(0, 0)]
        copies_out = [None, None]
        for ci in range(n_chunks):
            slot = ci % 2
            if ci + 1 < n_chunks:
                copies_in.append(start_in(ci + 1, (ci + 1) % 2))
            copies_in[ci].wait()
            if copies_out[slot] is not None:
                copies_out[slot].wait()
            s_c = s_v.at[slot]
            o_c = o_v.at[slot]

            @pl.loop(0, C)
            def _(t):
                s0 = s_c[t, 0:16]
                s1 = s_c[t, 16:32]
                s2 = s_c[t, 32:48]
                s3 = s_c[t, 48:64]
                v0 = jnp.sort(s0)
                v1 = jnp.sort(s1)
                v2 = jnp.sort(s2)
                v3 = jnp.sort(s3)
                lo01 = _merge_lo(v0, v1)
                hi01 = _merge_hi(v0, v1)
                lo23 = _merge_lo(v2, v3)
                hi23 = _merge_hi(v2, v3)
                h1 = _merge_hi(lo01, lo23)
                l2 = _merge_lo(h1, hi01)
                l3 = _merge_lo(l2, hi23)
                thr = l3[n_drop - 16]
                e0 = jnp.where(s0 >= thr, jnp.exp(s0), 0.0)
                e1 = jnp.where(s1 >= thr, jnp.exp(s1), 0.0)
                e2 = jnp.where(s2 >= thr, jnp.exp(s2), 0.0)
                e3 = jnp.where(s3 >= thr, jnp.exp(s3), 0.0)
                tot = jnp.sum(e0 + e1 + e2 + e3, axis=0)
                o_c[t, 0:16] = e0 / tot
                o_c[t, 16:32] = e1 / tot
                o_c[t, 32:48] = e2 / tot
                o_c[t, 48:64] = e3 / tot

            copies_out[slot] = start_out(ci, slot)

        for cp_out in copies_out:
            if cp_out is not None:
                cp_out.wait()

    return route


def kernel(x, W):
    B, S, H = x.shape
    E = W.shape[0]
    N = B * S
    n_drop = E - max(1, int(E * _ACTIVE_RATIO))
    T = 512
    xf = x.reshape(N, H)

    scores = pl.pallas_call(
        _gate_block,
        grid=(N // T,),
        in_specs=[
            pl.BlockSpec((T, H), lambda i: (i, 0)),
            pl.BlockSpec((E, H), lambda i: (0, 0)),
        ],
        out_specs=pl.BlockSpec((T, E), lambda i: (i, 0)),
        out_shape=jax.ShapeDtypeStruct((N, E), jnp.float32),
        compiler_params=pltpu.CompilerParams(
            dimension_semantics=("arbitrary",),
        ),
    )(xf, W)

    return _sc_route(N, E, 256, n_drop)(scores).reshape(B, S, E)
